# baseline (speedup 1.0000x reference)
import functools

import jax
import jax.numpy as jnp
from jax import lax
from jax.experimental import pallas as pl
from jax.experimental.pallas import tpu as pltpu
from jax.experimental.pallas import tpu_sc as plsc


_B, _C, _H, _W = 8, 1024, 24, 24
_HW = _H * _W
_ROWS = _B * _HW
_NW = 32
_RPW = _ROWS // _NW
_SCCH = 48
_NK = _C // 16
_RC = 2304
_BIG = 1 << 20


@functools.partial(
    pl.kernel,
    mesh=plsc.VectorSubcoreMesh(core_axis_name="c", subcore_axis_name="s"),
    out_type=jax.ShapeDtypeStruct((_ROWS,), jnp.int32),
    scratch_types=[
        pltpu.VMEM((_SCCH, _C), jnp.float32),
        pltpu.VMEM((_RPW,), jnp.int32),
    ],
)
def _sc_argmax(x_hbm, out_hbm, buf, res):
    wid = lax.axis_index("s") * 2 + lax.axis_index("c")
    base = wid * _RPW
    for ch in range(_RPW // _SCCH):
        pltpu.sync_copy(x_hbm.at[pl.ds(base + ch * _SCCH, _SCCH)], buf)

        def rgroup_body(g, _):
            def row_body(r16, acc):
                row = g * 16 + r16

                def k_body(k, carry):
                    m, idx = carry
                    v = buf[row, pl.ds(16 * k, 16)]
                    take = v > m
                    lanes = lax.iota(jnp.int32, 16) + 16 * k
                    return (jnp.where(take, v, m),
                            jnp.where(take, lanes, idx))

                m0 = buf[row, pl.ds(0, 16)]
                i0 = lax.iota(jnp.int32, 16)
                m, idx = lax.fori_loop(1, _NK, k_body, (m0, i0))
                rmax = m
                for sh in (8, 4, 2, 1):
                    perm = lax.iota(jnp.int32, 16) ^ sh
                    rmax = jnp.maximum(
                        rmax, rmax.at[perm].get(mode="promise_in_bounds"))
                cand = jnp.where(m == rmax, idx, _BIG)
                for sh in (8, 4, 2, 1):
                    perm = lax.iota(jnp.int32, 16) ^ sh
                    cand = jnp.minimum(
                        cand, cand.at[perm].get(mode="promise_in_bounds"))
                return jnp.where(lax.iota(jnp.int32, 16) == r16, cand, acc)

            acc = lax.fori_loop(0, 16, row_body,
                                jnp.zeros((16,), jnp.int32))
            res[pl.ds(ch * _SCCH + g * 16, 16)] = acc
            return 0

        lax.fori_loop(0, _SCCH // 16, rgroup_body, 0)
    pltpu.sync_copy(res, out_hbm.at[pl.ds(base, _RPW)])


def _copy_body(x_ref, xo_ref):
    xo_ref[...] = x_ref[...]


def kernel(inputs):
    xt = inputs.transpose(0, 2, 3, 1).reshape(_ROWS, _C)
    idx = _sc_argmax(xt)
    x_out = pl.pallas_call(
        _copy_body,
        grid=(_ROWS // _RC,),
        in_specs=[pl.BlockSpec((_RC, _C), lambda i: (i, 0))],
        out_specs=pl.BlockSpec((_RC, _C), lambda i: (i, 0)),
        out_shape=jax.ShapeDtypeStruct((_ROWS, _C), jnp.float32),
    )(xt)
    x_out = x_out.reshape(_B, _H, _W, _C).transpose(0, 3, 1, 2)
    return (x_out, idx.reshape(_B, 1, _H, _W))


# speedup vs baseline: 5.1275x; 5.1275x over previous
import jax
import jax.numpy as jnp
from jax import lax
from jax.experimental import pallas as pl


_B, _C, _H, _W = 8, 1024, 24, 24
_HW = _H * _W
_ROWS = _B * _HW
_NT = _C // 128
_RC = 2304
_NS = _ROWS // _RC
_BIG = 1 << 20


def _body(x_ref, xo_ref, idx_ref):
    x = x_ref[...]
    xo_ref[...] = x
    m = x[:, 0:128]
    tidx = jnp.zeros((_RC, 128), jnp.int32)
    for t in range(1, _NT):
        xt = x[:, 128 * t:128 * (t + 1)]
        gt = xt > m
        m = jnp.where(gt, xt, m)
        tidx = jnp.where(gt, t, tidx)
    rowmax = jnp.max(m, axis=1, keepdims=True)
    lane = lax.broadcasted_iota(jnp.int32, (_RC, 128), 1)
    cand = jnp.where(m == rowmax, 128 * tidx + lane, _BIG)
    idx_ref[0, 0] = jnp.min(cand, axis=1)


def kernel(inputs):
    xt = inputs.transpose(0, 2, 3, 1).reshape(_ROWS, _C)
    x_out, idx = pl.pallas_call(
        _body,
        grid=(_NS,),
        in_specs=[pl.BlockSpec((_RC, _C), lambda i: (i, 0))],
        out_specs=[
            pl.BlockSpec((_RC, _C), lambda i: (i, 0)),
            pl.BlockSpec((1, 1, _RC), lambda i: (i, 0, 0)),
        ],
        out_shape=[
            jax.ShapeDtypeStruct((_ROWS, _C), jnp.float32),
            jax.ShapeDtypeStruct((_NS, 1, _RC), jnp.int32),
        ],
    )(xt)
    x_out = x_out.reshape(_B, _H, _W, _C).transpose(0, 3, 1, 2)
    return (x_out, idx.reshape(_B, 1, _H, _W))
